# baseline (speedup 1.0000x reference)
import functools

import jax
import jax.numpy as jnp
from jax import lax
from jax.experimental import pallas as pl
from jax.experimental.pallas import tpu as pltpu
from jax.experimental.pallas import tpu_sc as plsc

N = 100000
D = 128
B = 256
H = 4

RB = 4096
NB = -(-N // RB)
NPAD = NB * RB
RB2 = 8192
NB2 = -(-N // RB2)
NPAD2 = NB2 * RB2
SLEN = 256
T = 128
GT = 160
NSEG = B // 32
NC = 2
NS = 16


def _gate_body(x_ref, wt_ref, bias_ref, gate_ref):
    x = x_ref[...]
    logitst = lax.dot_general(
        wt_ref[...], x, (((1,), (1,)), ((), ())),
        preferred_element_type=jnp.float32)
    sigt = jax.nn.sigmoid(logitst + bias_ref[...])
    gate = jnp.sum(sigt, axis=0) * (1.0 / H) - 0.5
    gate_ref[0, 0, :] = gate


def _starts_body(ids_ref, svals_ref, starts_ref):
    i = pl.program_id(0)
    ids = ids_ref[0, 0, :]
    sv = svals_ref[...]
    one = jnp.ones((), jnp.bfloat16)
    hi = jnp.floor(ids * jnp.bfloat16(0.0625))
    lo = ids - jnp.bfloat16(16) * hi
    cmph = jnp.where(hi[None, :] < sv, one, 0 * one)
    eqh = jnp.where(hi[None, :] == sv, one, 0 * one)
    cmpl = jnp.where(lo[None, :] < sv, one, 0 * one)
    chb = jnp.dot(cmph, jnp.ones((RB2, 16), jnp.bfloat16),
                  preferred_element_type=jnp.float32)
    m = lax.dot_general(eqh, cmpl, (((1,), (1,)), ((), ())),
                        preferred_element_type=jnp.float32)
    cnt = chb + m

    @pl.when(i == 0)
    def _():
        starts_ref[0] = cnt

    @pl.when(i > 0)
    def _():
        starts_ref[0] += cnt


def _tc_stage(feats, wt, bias, ids3, svals):
    gate3 = pl.pallas_call(
        _gate_body,
        grid=(NB,),
        in_specs=[
            pl.BlockSpec((RB, D), lambda i: (i, 0)),
            pl.BlockSpec((8, D), lambda i: (0, 0)),
            pl.BlockSpec((8, RB), lambda i: (0, 0)),
        ],
        out_specs=pl.BlockSpec((1, 1, RB), lambda i: (i, 0, 0)),
        out_shape=jax.ShapeDtypeStruct((NB, 1, RB), jnp.float32),
        compiler_params=pltpu.CompilerParams(
            dimension_semantics=("arbitrary",)),
    )(feats, wt, bias)
    starts3 = pl.pallas_call(
        _starts_body,
        grid=(NB2,),
        in_specs=[
            pl.BlockSpec((1, 1, RB2), lambda i: (i, 0, 0)),
            pl.BlockSpec((16, 1), lambda i: (0, 0)),
        ],
        out_specs=pl.BlockSpec((1, 16, 16), lambda i: (0, 0, 0)),
        out_shape=jax.ShapeDtypeStruct((1, 16, 16), jnp.float32),
    )(ids3, svals)
    return gate3.reshape(NPAD), starts3.reshape(SLEN)


def _sc_body(feats_hbm, gate_hbm, starts_hbm, out_hbm,
             startbuf, rowbuf, gatebuf, rsem, gsem, outbuf):
    wid = lax.axis_index("s") * NC + lax.axis_index("c")
    seg0 = wid * NSEG
    pltpu.sync_copy(starts_hbm.at[pl.ds(seg0, 16)], startbuf)
    sv = startbuf[...].astype(jnp.int32)

    def issue(r0, t, slot):
        a = r0 + t * T
        fa = jnp.minimum(a, N - T)
        ga = (a // 8) * 8
        pltpu.make_async_copy(
            feats_hbm.at[pl.ds(fa, T)], rowbuf.at[slot], rsem).start()
        pltpu.make_async_copy(
            gate_hbm.at[pl.ds(ga, GT)], gatebuf.at[slot, pl.ds(0, GT)],
            gsem).start()

    def wait_tile():
        pltpu.make_async_copy(
            feats_hbm.at[pl.ds(0, T)], rowbuf.at[0], rsem).wait()
        pltpu.make_async_copy(
            gate_hbm.at[pl.ds(0, GT)], gatebuf.at[0, pl.ds(0, GT)],
            gsem).wait()

    for i in range(NSEG):
        r0 = sv[i]
        r1 = sv[i + 1]
        cnt = r1 - r0
        nt = (cnt + (T - 1)) // T

        @pl.when(nt > 0)
        def _():
            issue(r0, 0, 0)

        def tile_body(t, accs):
            slot = lax.rem(t, 2)

            @pl.when(t + 1 < nt)
            def _():
                issue(r0, t + 1, 1 - slot)

            wait_tile()
            a = r0 + t * T
            d = a - jnp.minimum(a, N - T)
            off = a - (a // 8) * 8
            nrows = jnp.minimum(cnt - t * T, T)

            def row_body2(j, accs2):
                s0, s1, m0, m1 = accs2
                r = 2 * j
                g0 = gatebuf[slot, pl.ds(r + off, 16)][0]
                g1 = gatebuf[slot, pl.ds(r + 1 + off, 16)][0]
                ns0, ns1, nm0, nm1 = [], [], [], []
                for k in range(8):
                    v0 = rowbuf[slot, r + d, pl.ds(k * 16, 16)]
                    v1 = rowbuf[slot, r + 1 + d, pl.ds(k * 16, 16)]
                    ns0.append(s0[k] + g0 * v0)
                    nm0.append(jnp.maximum(m0[k], v0))
                    ns1.append(s1[k] + g1 * v1)
                    nm1.append(jnp.maximum(m1[k], v1))
                return (tuple(ns0), tuple(ns1), tuple(nm0), tuple(nm1))

            s0, s1, m0, m1 = lax.fori_loop(0, nrows // 2, row_body2, accs)

            odd = lax.rem(nrows, 2) == 1
            rl = nrows - 1
            gl = jnp.where(odd, gatebuf[slot, pl.ds(rl + off, 16)][0], 0.0)
            ns0, nm0 = [], []
            for k in range(8):
                v = rowbuf[slot, rl + d, pl.ds(k * 16, 16)]
                ns0.append(s0[k] + gl * v)
                nm0.append(jnp.maximum(
                    m0[k], jnp.where(odd, v, -jnp.inf)))
            return (tuple(ns0), s1, tuple(nm0), m1)

        zero = jnp.zeros((16,), jnp.float32)
        ninf = jnp.full((16,), -jnp.inf, jnp.float32)
        s0, s1, m0, m1 = lax.fori_loop(
            0, nt, tile_body,
            ((zero,) * 8, (zero,) * 8, (ninf,) * 8, (ninf,) * 8))
        for k in range(8):
            outbuf[i, pl.ds(k * 16, 16)] = s0[k] + s1[k]
            outbuf[i, pl.ds(D + k * 16, 16)] = jnp.maximum(m0[k], m1[k])

    pltpu.sync_copy(outbuf, out_hbm.at[pl.ds(seg0, NSEG)])


_sc_stage = functools.partial(
    pl.kernel,
    out_type=jax.ShapeDtypeStruct((B, 2 * D), jnp.float32),
    mesh=plsc.VectorSubcoreMesh(core_axis_name="c", subcore_axis_name="s"),
    compiler_params=pltpu.CompilerParams(use_tc_tiling_on_sc=False),
    scratch_types=[
        pltpu.VMEM((16,), jnp.float32),
        pltpu.VMEM((2, T, D), jnp.float32),
        pltpu.VMEM((2, GT + 16), jnp.float32),
        pltpu.SemaphoreType.DMA,
        pltpu.SemaphoreType.DMA,
        pltpu.VMEM((NSEG, 2 * D), jnp.float32),
    ],
)(_sc_body)


def kernel(feats, segment_ids, W, b):
    ids3 = jnp.concatenate(
        [segment_ids.astype(jnp.bfloat16),
         jnp.full((NPAD2 - N,), 512.0, jnp.bfloat16)]
    ).reshape(NB2, 1, RB2)
    svals = jnp.arange(16, dtype=jnp.float32).astype(
        jnp.bfloat16).reshape(16, 1)
    wt = jnp.pad(W, ((0, 8 - H), (0, 0)))
    bias = jnp.broadcast_to(jnp.pad(b, (0, 8 - H))[:, None], (8, RB))
    gate, starts = _tc_stage(feats, wt, bias, ids3, svals)
    starts_full = jnp.concatenate(
        [starts, jnp.full((8,), float(N), jnp.float32)])
    return _sc_stage(feats, gate, starts_full)


# speedup vs baseline: 1.0087x; 1.0087x over previous
import functools

import jax
import jax.numpy as jnp
from jax import lax
from jax.experimental import pallas as pl
from jax.experimental.pallas import tpu as pltpu
from jax.experimental.pallas import tpu_sc as plsc

N = 100000
D = 128
B = 256
H = 4

RB = 4096
NB = -(-N // RB)
NPAD = NB * RB
RB2 = 8192
NB2 = -(-N // RB2)
NPAD2 = NB2 * RB2
SLEN = 256
T = 256
GT = 288
NSEG = B // 32
NC = 2
NS = 16


def _gate_body(x_ref, wt_ref, bias_ref, gate_ref):
    x = x_ref[...]
    logitst = lax.dot_general(
        wt_ref[...], x, (((1,), (1,)), ((), ())),
        preferred_element_type=jnp.float32)
    sigt = jax.nn.sigmoid(logitst + bias_ref[...])
    gate = jnp.sum(sigt, axis=0) * (1.0 / H) - 0.5
    gate_ref[0, 0, :] = gate


def _starts_body(ids_ref, svals_ref, starts_ref):
    i = pl.program_id(0)
    ids = ids_ref[0, 0, :]
    sv = svals_ref[...]
    one = jnp.ones((), jnp.bfloat16)
    hi = jnp.floor(ids * jnp.bfloat16(0.0625))
    lo = ids - jnp.bfloat16(16) * hi
    cmph = jnp.where(hi[None, :] < sv, one, 0 * one)
    eqh = jnp.where(hi[None, :] == sv, one, 0 * one)
    cmpl = jnp.where(lo[None, :] < sv, one, 0 * one)
    chb = jnp.dot(cmph, jnp.ones((RB2, 16), jnp.bfloat16),
                  preferred_element_type=jnp.float32)
    m = lax.dot_general(eqh, cmpl, (((1,), (1,)), ((), ())),
                        preferred_element_type=jnp.float32)
    cnt = chb + m

    @pl.when(i == 0)
    def _():
        starts_ref[0] = cnt

    @pl.when(i > 0)
    def _():
        starts_ref[0] += cnt


def _tc_stage(feats, wt, bias, ids3, svals):
    gate3 = pl.pallas_call(
        _gate_body,
        grid=(NB,),
        in_specs=[
            pl.BlockSpec((RB, D), lambda i: (i, 0)),
            pl.BlockSpec((8, D), lambda i: (0, 0)),
            pl.BlockSpec((8, RB), lambda i: (0, 0)),
        ],
        out_specs=pl.BlockSpec((1, 1, RB), lambda i: (i, 0, 0)),
        out_shape=jax.ShapeDtypeStruct((NB, 1, RB), jnp.float32),
        compiler_params=pltpu.CompilerParams(
            dimension_semantics=("arbitrary",)),
    )(feats, wt, bias)
    starts3 = pl.pallas_call(
        _starts_body,
        grid=(NB2,),
        in_specs=[
            pl.BlockSpec((1, 1, RB2), lambda i: (i, 0, 0)),
            pl.BlockSpec((16, 1), lambda i: (0, 0)),
        ],
        out_specs=pl.BlockSpec((1, 16, 16), lambda i: (0, 0, 0)),
        out_shape=jax.ShapeDtypeStruct((1, 16, 16), jnp.float32),
    )(ids3, svals)
    return gate3.reshape(NPAD), starts3.reshape(SLEN)


def _sc_body(feats_hbm, gate_hbm, starts_hbm, out_hbm,
             startbuf, rowbuf, gatebuf, rsem, gsem, outbuf):
    wid = lax.axis_index("s") * NC + lax.axis_index("c")
    seg0 = wid * NSEG
    pltpu.sync_copy(starts_hbm.at[pl.ds(seg0, 16)], startbuf)
    sv = startbuf[...].astype(jnp.int32)

    def issue(r0, t, slot):
        a = r0 + t * T
        fa = jnp.minimum(a, N - T)
        ga = (a // 8) * 8
        pltpu.make_async_copy(
            feats_hbm.at[pl.ds(fa, T)], rowbuf.at[slot], rsem).start()
        pltpu.make_async_copy(
            gate_hbm.at[pl.ds(ga, GT)], gatebuf.at[slot, pl.ds(0, GT)],
            gsem).start()

    def wait_tile():
        pltpu.make_async_copy(
            feats_hbm.at[pl.ds(0, T)], rowbuf.at[0], rsem).wait()
        pltpu.make_async_copy(
            gate_hbm.at[pl.ds(0, GT)], gatebuf.at[0, pl.ds(0, GT)],
            gsem).wait()

    for i in range(NSEG):
        r0 = sv[i]
        r1 = sv[i + 1]
        cnt = r1 - r0
        nt = (cnt + (T - 1)) // T

        @pl.when(nt > 0)
        def _():
            issue(r0, 0, 0)

        def tile_body(t, accs):
            slot = lax.rem(t, 2)

            @pl.when(t + 1 < nt)
            def _():
                issue(r0, t + 1, 1 - slot)

            wait_tile()
            a = r0 + t * T
            d = a - jnp.minimum(a, N - T)
            off = a - (a // 8) * 8
            nrows = jnp.minimum(cnt - t * T, T)

            def row_body(r, accs2):
                sums, maxs = accs2
                g = gatebuf[slot, pl.ds(r + off, 16)][0]
                ns, nm = [], []
                for k in range(8):
                    v = rowbuf[slot, r + d, pl.ds(k * 16, 16)]
                    ns.append(sums[k] + g * v)
                    nm.append(jnp.maximum(maxs[k], v))
                return (tuple(ns), tuple(nm))

            return lax.fori_loop(0, nrows, row_body, accs)

        zero = jnp.zeros((16,), jnp.float32)
        ninf = jnp.full((16,), -jnp.inf, jnp.float32)
        sums, maxs = lax.fori_loop(
            0, nt, tile_body, ((zero,) * 8, (ninf,) * 8))
        for k in range(8):
            outbuf[i, pl.ds(k * 16, 16)] = sums[k]
            outbuf[i, pl.ds(D + k * 16, 16)] = maxs[k]

    pltpu.sync_copy(outbuf, out_hbm.at[pl.ds(seg0, NSEG)])


_sc_stage = functools.partial(
    pl.kernel,
    out_type=jax.ShapeDtypeStruct((B, 2 * D), jnp.float32),
    mesh=plsc.VectorSubcoreMesh(core_axis_name="c", subcore_axis_name="s"),
    compiler_params=pltpu.CompilerParams(use_tc_tiling_on_sc=False),
    scratch_types=[
        pltpu.VMEM((16,), jnp.float32),
        pltpu.VMEM((2, T, D), jnp.float32),
        pltpu.VMEM((2, GT + 16), jnp.float32),
        pltpu.SemaphoreType.DMA,
        pltpu.SemaphoreType.DMA,
        pltpu.VMEM((NSEG, 2 * D), jnp.float32),
    ],
)(_sc_body)


def kernel(feats, segment_ids, W, b):
    ids3 = jnp.concatenate(
        [segment_ids.astype(jnp.bfloat16),
         jnp.full((NPAD2 - N,), 512.0, jnp.bfloat16)]
    ).reshape(NB2, 1, RB2)
    svals = jnp.arange(16, dtype=jnp.float32).astype(
        jnp.bfloat16).reshape(16, 1)
    wt = jnp.pad(W, ((0, 8 - H), (0, 0)))
    bias = jnp.broadcast_to(jnp.pad(b, (0, 8 - H))[:, None], (8, RB))
    gate, starts = _tc_stage(feats, wt, bias, ids3, svals)
    starts_full = jnp.concatenate(
        [starts, jnp.full((8,), float(N), jnp.float32)])
    return _sc_stage(feats, gate, starts_full)


# speedup vs baseline: 1.0972x; 1.0877x over previous
import functools

import jax
import jax.numpy as jnp
from jax import lax
from jax.experimental import pallas as pl
from jax.experimental.pallas import tpu as pltpu
from jax.experimental.pallas import tpu_sc as plsc

N = 100000
D = 128
B = 256
H = 4

RB = 4096
NB = -(-N // RB)
NPAD = NB * RB
RB2 = 8192
NB2 = -(-N // RB2)
NPAD2 = NB2 * RB2
SLEN = 256
T = 256
GT = 288
NSEG = B // 32
NC = 2
NS = 16


def _gate_body(x_ref, wt_ref, bias_ref, ids_ref, svals_ref,
               gate_ref, starts_ref):
    i = pl.program_id(0)
    x = x_ref[...]
    logitst = lax.dot_general(
        wt_ref[...], x, (((1,), (1,)), ((), ())),
        preferred_element_type=jnp.float32)
    sigt = jax.nn.sigmoid(logitst + bias_ref[...])
    gate = jnp.sum(sigt, axis=0) * (1.0 / H) - 0.5
    gate_ref[0, 0, :] = gate

    ids = ids_ref[0, 0, :]
    sv = svals_ref[...]
    one = jnp.ones((), jnp.bfloat16)
    hi = jnp.floor(ids * jnp.bfloat16(0.0625))
    lo = ids - jnp.bfloat16(16) * hi
    cmph = jnp.where(hi[None, :] < sv, one, 0 * one)
    eqh = jnp.where(hi[None, :] == sv, one, 0 * one)
    cmpl = jnp.where(lo[None, :] < sv, one, 0 * one)
    chb = jnp.dot(cmph, jnp.ones((RB, 16), jnp.bfloat16),
                  preferred_element_type=jnp.float32)
    m = lax.dot_general(eqh, cmpl, (((1,), (1,)), ((), ())),
                        preferred_element_type=jnp.float32)
    cnt = chb + m

    @pl.when(i == 0)
    def _():
        starts_ref[0] = cnt

    @pl.when(i > 0)
    def _():
        starts_ref[0] += cnt


def _tc_stage(feats, wt, bias, ids3, svals):
    gate3, starts3 = pl.pallas_call(
        _gate_body,
        grid=(NB,),
        in_specs=[
            pl.BlockSpec((RB, D), lambda i: (i, 0)),
            pl.BlockSpec((8, D), lambda i: (0, 0)),
            pl.BlockSpec((8, RB), lambda i: (0, 0)),
            pl.BlockSpec((1, 1, RB), lambda i: (i, 0, 0)),
            pl.BlockSpec((16, 1), lambda i: (0, 0)),
        ],
        out_specs=[
            pl.BlockSpec((1, 1, RB), lambda i: (i, 0, 0)),
            pl.BlockSpec((1, 16, 16), lambda i: (0, 0, 0)),
        ],
        out_shape=[
            jax.ShapeDtypeStruct((NB, 1, RB), jnp.float32),
            jax.ShapeDtypeStruct((1, 16, 16), jnp.float32),
        ],
        compiler_params=pltpu.CompilerParams(
            dimension_semantics=("arbitrary",)),
    )(feats, wt, bias, ids3, svals)
    return gate3.reshape(NPAD), starts3.reshape(SLEN)


def _sc_body(feats_hbm, gate_hbm, starts_hbm, out_hbm,
             startbuf, rowbuf, gatebuf, rsem, gsem, outbuf):
    wid = lax.axis_index("s") * NC + lax.axis_index("c")
    seg0 = wid * NSEG
    pltpu.sync_copy(starts_hbm.at[pl.ds(seg0, 16)], startbuf)
    sv = startbuf[...].astype(jnp.int32)

    def issue(r0, t, slot):
        a = r0 + t * T
        fa = jnp.minimum(a, N - T)
        ga = (a // 8) * 8
        pltpu.make_async_copy(
            feats_hbm.at[pl.ds(fa, T)], rowbuf.at[slot], rsem).start()
        pltpu.make_async_copy(
            gate_hbm.at[pl.ds(ga, GT)], gatebuf.at[slot, pl.ds(0, GT)],
            gsem).start()

    def wait_tile():
        pltpu.make_async_copy(
            feats_hbm.at[pl.ds(0, T)], rowbuf.at[0], rsem).wait()
        pltpu.make_async_copy(
            gate_hbm.at[pl.ds(0, GT)], gatebuf.at[0, pl.ds(0, GT)],
            gsem).wait()

    for i in range(NSEG):
        r0 = sv[i]
        r1 = sv[i + 1]
        cnt = r1 - r0
        nt = (cnt + (T - 1)) // T

        @pl.when(nt > 0)
        def _():
            issue(r0, 0, 0)

        def tile_body(t, accs):
            slot = lax.rem(t, 2)

            @pl.when(t + 1 < nt)
            def _():
                issue(r0, t + 1, 1 - slot)

            wait_tile()
            a = r0 + t * T
            d = a - jnp.minimum(a, N - T)
            off = a - (a // 8) * 8
            nrows = jnp.minimum(cnt - t * T, T)

            def row_body(r, accs2):
                sums, maxs = accs2
                g = gatebuf[slot, pl.ds(r + off, 16)][0]
                ns, nm = [], []
                for k in range(8):
                    v = rowbuf[slot, r + d, pl.ds(k * 16, 16)]
                    ns.append(sums[k] + g * v)
                    nm.append(jnp.maximum(maxs[k], v))
                return (tuple(ns), tuple(nm))

            return lax.fori_loop(0, nrows, row_body, accs)

        zero = jnp.zeros((16,), jnp.float32)
        ninf = jnp.full((16,), -jnp.inf, jnp.float32)
        sums, maxs = lax.fori_loop(
            0, nt, tile_body, ((zero,) * 8, (ninf,) * 8))
        for k in range(8):
            outbuf[i, pl.ds(k * 16, 16)] = sums[k]
            outbuf[i, pl.ds(D + k * 16, 16)] = maxs[k]

    pltpu.sync_copy(outbuf, out_hbm.at[pl.ds(seg0, NSEG)])


_sc_stage = functools.partial(
    pl.kernel,
    out_type=jax.ShapeDtypeStruct((B, 2 * D), jnp.float32),
    mesh=plsc.VectorSubcoreMesh(core_axis_name="c", subcore_axis_name="s"),
    compiler_params=pltpu.CompilerParams(use_tc_tiling_on_sc=False),
    scratch_types=[
        pltpu.VMEM((16,), jnp.float32),
        pltpu.VMEM((2, T, D), jnp.float32),
        pltpu.VMEM((2, GT + 16), jnp.float32),
        pltpu.SemaphoreType.DMA,
        pltpu.SemaphoreType.DMA,
        pltpu.VMEM((NSEG, 2 * D), jnp.float32),
    ],
)(_sc_body)


def kernel(feats, segment_ids, W, b):
    ids3 = jnp.concatenate(
        [segment_ids.astype(jnp.bfloat16),
         jnp.full((NPAD - N,), 512.0, jnp.bfloat16)]
    ).reshape(NB, 1, RB)
    svals = jnp.arange(16, dtype=jnp.float32).astype(
        jnp.bfloat16).reshape(16, 1)
    wt = jnp.pad(W, ((0, 8 - H), (0, 0)))
    bias = jnp.broadcast_to(jnp.pad(b, (0, 8 - H))[:, None], (8, RB))
    gate, starts = _tc_stage(feats, wt, bias, ids3, svals)
    starts_full = jnp.concatenate(
        [starts, jnp.full((8,), float(N), jnp.float32)])
    return _sc_stage(feats, gate, starts_full)
